# initial kernel scaffold (unmeasured)
import jax
import jax.numpy as jnp
from jax import lax
from jax.experimental import pallas as pl
from jax.experimental.pallas import tpu as pltpu

N_DEV = 16


def kernel(x, w_mat):
    M, K = x.shape
    _, N = w_mat.shape
    CH = M // N_DEV
    N_STEPS = 2 * (N_DEV - 1)

    def body(x_ref, w_ref, out_ref, comm, stage, send_sems, recv_sems,
             credit_sem, out_sems):
        me = lax.axis_index("i")
        left = (me - 1) % N_DEV
        right = (me + 1) % N_DEV

        barrier = pltpu.get_barrier_semaphore()
        for nbr in (left, right):
            pl.semaphore_signal(barrier, inc=1, device_id=(nbr,),
                                device_id_type=pl.DeviceIdType.MESH)
        pl.semaphore_wait(barrier, 2)

        def partial_chunk(c):
            xc = x_ref[pl.ds(c * CH, CH), :]
            return jnp.dot(xc, w_ref[:, :], preferred_element_type=jnp.float32)

        comm[0, :, :] = partial_chunk(me).astype(jnp.bfloat16)

        pending = [None, None]
        store_i = [0]

        def relu_store(slot, c):
            st = store_i[0] % 2
            if pending[st] is not None:
                pending[st].wait()
            stage[st, :, :] = jnp.maximum(comm[slot, :, :].astype(jnp.float32), 0.0)
            cp = pltpu.make_async_copy(
                stage.at[st], out_ref.at[pl.ds(c * CH, CH), :], out_sems.at[st])
            cp.start()
            pending[st] = cp
            store_i[0] += 1

        for u in range(N_STEPS):
            s_slot = u % 2
            r_slot = (u + 1) % 2
            if u >= 1:
                pl.semaphore_wait(credit_sem, 1)
            rdma = pltpu.make_async_remote_copy(
                src_ref=comm.at[s_slot],
                dst_ref=comm.at[r_slot],
                send_sem=send_sems.at[s_slot],
                recv_sem=recv_sems.at[r_slot],
                device_id=(right,),
                device_id_type=pl.DeviceIdType.MESH,
            )
            rdma.start()
            if u == N_DEV - 1:
                relu_store(s_slot, (me + 1) % N_DEV)
            rdma.wait_send()
            if u < N_STEPS - 1:
                pl.semaphore_signal(credit_sem, inc=1, device_id=(left,),
                                    device_id_type=pl.DeviceIdType.MESH)
            rdma.wait_recv()
            if u < N_DEV - 1:
                c = (me - u - 1) % N_DEV
                comm[r_slot, :, :] = (
                    comm[r_slot, :, :].astype(jnp.float32) + partial_chunk(c)
                ).astype(jnp.bfloat16)
            else:
                t = u - (N_DEV - 1)
                c = (me - t) % N_DEV
                relu_store(r_slot, c)

        for st in range(2):
            if pending[st] is not None:
                pending[st].wait()

    return pl.pallas_call(
        body,
        out_shape=jax.ShapeDtypeStruct((M, N), jnp.float32),
        in_specs=[
            pl.BlockSpec(memory_space=pltpu.VMEM),
            pl.BlockSpec(memory_space=pltpu.VMEM),
        ],
        out_specs=pl.BlockSpec(memory_space=pltpu.ANY),
        scratch_shapes=[
            pltpu.VMEM((2, CH, N), jnp.bfloat16),
            pltpu.VMEM((2, CH, N), jnp.float32),
            pltpu.SemaphoreType.DMA((2,)),
            pltpu.SemaphoreType.DMA((2,)),
            pltpu.SemaphoreType.REGULAR,
            pltpu.SemaphoreType.DMA((2,)),
        ],
        compiler_params=pltpu.CompilerParams(collective_id=0),
    )(x, w_mat)


# baseline (device time: 1551010 ns/iter reference)
import jax
import jax.numpy as jnp
from jax import lax
from jax.experimental import pallas as pl
from jax.experimental.pallas import tpu as pltpu

N_DEV = 16


def kernel(x, w_mat):
    M, K = x.shape
    _, N = w_mat.shape
    CH = M // N_DEV
    N_STEPS = 2 * (N_DEV - 1)

    def body(x_ref, w_ref, out_ref, comm, stage, send_sems, recv_sems,
             credit_sem, out_sems):
        me = lax.axis_index("i")
        left = (me - 1) % N_DEV
        right = (me + 1) % N_DEV

        barrier = pltpu.get_barrier_semaphore()
        for nbr in (left, right):
            pl.semaphore_signal(barrier, inc=1, device_id=(nbr,),
                                device_id_type=pl.DeviceIdType.MESH)
        pl.semaphore_wait(barrier, 2)

        def partial_chunk(c):
            xc = x_ref[pl.ds(c * CH, CH), :]
            return jnp.dot(xc, w_ref[:, :], preferred_element_type=jnp.float32)

        comm[0, :, :] = partial_chunk(me).astype(jnp.bfloat16)

        pending = [None, None]
        store_i = [0]

        def relu_store(slot, c):
            st = store_i[0] % 2
            if pending[st] is not None:
                pending[st].wait()
            stage[st, :, :] = jnp.maximum(comm[slot, :, :].astype(jnp.float32), 0.0)
            cp = pltpu.make_async_copy(
                stage.at[st], out_ref.at[pl.ds(c * CH, CH), :], out_sems.at[st])
            cp.start()
            pending[st] = cp
            store_i[0] += 1

        for u in range(N_STEPS):
            s_slot = u % 2
            r_slot = (u + 1) % 2
            if u >= 1:
                pl.semaphore_wait(credit_sem, 1)
            rdma = pltpu.make_async_remote_copy(
                src_ref=comm.at[s_slot],
                dst_ref=comm.at[r_slot],
                send_sem=send_sems.at[s_slot],
                recv_sem=recv_sems.at[r_slot],
                device_id=(right,),
                device_id_type=pl.DeviceIdType.MESH,
            )
            rdma.start()
            if u == N_DEV - 1:
                relu_store(s_slot, (me + 1) % N_DEV)
            rdma.wait_send()
            if u < N_STEPS - 1:
                pl.semaphore_signal(credit_sem, inc=1, device_id=(left,),
                                    device_id_type=pl.DeviceIdType.MESH)
            rdma.wait_recv()
            if u < N_DEV - 1:
                c = (me - u - 1) % N_DEV
                comm[r_slot, :, :] = (
                    comm[r_slot, :, :].astype(jnp.float32) + partial_chunk(c)
                ).astype(jnp.bfloat16)
            else:
                t = u - (N_DEV - 1)
                c = (me - t) % N_DEV
                relu_store(r_slot, c)

        for st in range(2):
            if pending[st] is not None:
                pending[st].wait()

    return pl.pallas_call(
        body,
        out_shape=jax.ShapeDtypeStruct((M, N), jnp.float32),
        in_specs=[
            pl.BlockSpec(memory_space=pltpu.VMEM),
            pl.BlockSpec(memory_space=pltpu.VMEM),
        ],
        out_specs=pl.BlockSpec(memory_space=pl.ANY),
        scratch_shapes=[
            pltpu.VMEM((2, CH, N), jnp.bfloat16),
            pltpu.VMEM((2, CH, N), jnp.float32),
            pltpu.SemaphoreType.DMA((2,)),
            pltpu.SemaphoreType.DMA((2,)),
            pltpu.SemaphoreType.REGULAR,
            pltpu.SemaphoreType.DMA((2,)),
        ],
        compiler_params=pltpu.CompilerParams(collective_id=0),
    )(x, w_mat)


# device time: 885176 ns/iter; 1.7522x vs baseline; 1.7522x over previous
import jax
import jax.numpy as jnp
from jax import lax
from jax.experimental import pallas as pl
from jax.experimental.pallas import tpu as pltpu

N_DEV = 16


def kernel(x, w_mat):
    M, K = x.shape
    _, N = w_mat.shape
    CH = M // N_DEV
    N2 = N // 2
    N_STEPS = 2 * (N_DEV - 1)

    def body(x_ref, w_ref, out_ref,
             comm_a, comm_b, stage_a, stage_b,
             send_a, recv_a, send_b, recv_b,
             credit_a, credit_b, osem_a, osem_b):
        me = lax.axis_index("i")
        left = (me - 1) % N_DEV
        right = (me + 1) % N_DEV

        barrier = pltpu.get_barrier_semaphore()
        for nbr in (left, right):
            pl.semaphore_signal(barrier, inc=1, device_id=(nbr,),
                                device_id_type=pl.DeviceIdType.MESH)
        pl.semaphore_wait(barrier, 2)

        def partial_half(c, half):
            xc = x_ref[pl.ds(c * CH, CH), :]
            wc = w_ref[:, pl.ds(half * N2, N2)]
            return jnp.dot(xc, wc, preferred_element_type=jnp.float32
                           ).astype(jnp.bfloat16)

        rings = [
            dict(comm=comm_a, stage=stage_a, send=send_a, recv=recv_a,
                 credit=credit_a, osem=osem_a, dst=right, ups=left,
                 sgn=1, half=0, pend=[None, None], n_store=[0]),
            dict(comm=comm_b, stage=stage_b, send=send_b, recv=recv_b,
                 credit=credit_b, osem=osem_b, dst=left, ups=right,
                 sgn=-1, half=1, pend=[None, None], n_store=[0]),
        ]

        for r in rings:
            r["comm"][0, :, :] = partial_half(me, r["half"])

        def relu_store(r, slot, c):
            st = r["n_store"][0] % 2
            if r["pend"][st] is not None:
                r["pend"][st].wait()
            r["stage"][st, :, :] = jnp.maximum(
                r["comm"][slot, :, :].astype(jnp.float32), 0.0)
            cp = pltpu.make_async_copy(
                r["stage"].at[st],
                out_ref.at[pl.ds(c * CH, CH), pl.ds(r["half"] * N2, N2)],
                r["osem"].at[st])
            cp.start()
            r["pend"][st] = cp
            r["n_store"][0] += 1

        for u in range(N_STEPS):
            s_slot = u % 2
            r_slot = (u + 1) % 2
            rdmas = []
            for r in rings:
                if u >= 1:
                    pl.semaphore_wait(r["credit"], 1)
                rdma = pltpu.make_async_remote_copy(
                    src_ref=r["comm"].at[s_slot],
                    dst_ref=r["comm"].at[r_slot],
                    send_sem=r["send"].at[s_slot],
                    recv_sem=r["recv"].at[r_slot],
                    device_id=(r["dst"],),
                    device_id_type=pl.DeviceIdType.MESH,
                )
                rdma.start()
                rdmas.append(rdma)

            padd = []
            if u < N_DEV - 1:
                for r in rings:
                    c = (me - r["sgn"] * (u + 1)) % N_DEV
                    padd.append(partial_half(c, r["half"]))
            elif u == N_DEV - 1:
                for r in rings:
                    relu_store(r, s_slot, (me + r["sgn"]) % N_DEV)

            for r, rdma in zip(rings, rdmas):
                rdma.wait_send()
                if u < N_STEPS - 1:
                    pl.semaphore_signal(r["credit"], inc=1,
                                        device_id=(r["ups"],),
                                        device_id_type=pl.DeviceIdType.MESH)
                rdma.wait_recv()

            if u < N_DEV - 1:
                for r, p in zip(rings, padd):
                    r["comm"][r_slot, :, :] = r["comm"][r_slot, :, :] + p
            else:
                t = u - (N_DEV - 1)
                for r in rings:
                    c = (me - r["sgn"] * t) % N_DEV
                    relu_store(r, r_slot, c)

        for r in rings:
            for st in range(2):
                if r["pend"][st] is not None:
                    r["pend"][st].wait()

    return pl.pallas_call(
        body,
        out_shape=jax.ShapeDtypeStruct((M, N), jnp.float32),
        in_specs=[
            pl.BlockSpec(memory_space=pltpu.VMEM),
            pl.BlockSpec(memory_space=pltpu.VMEM),
        ],
        out_specs=pl.BlockSpec(memory_space=pl.ANY),
        scratch_shapes=[
            pltpu.VMEM((2, CH, N2), jnp.bfloat16),
            pltpu.VMEM((2, CH, N2), jnp.bfloat16),
            pltpu.VMEM((2, CH, N2), jnp.float32),
            pltpu.VMEM((2, CH, N2), jnp.float32),
            pltpu.SemaphoreType.DMA((2,)),
            pltpu.SemaphoreType.DMA((2,)),
            pltpu.SemaphoreType.DMA((2,)),
            pltpu.SemaphoreType.DMA((2,)),
            pltpu.SemaphoreType.REGULAR,
            pltpu.SemaphoreType.REGULAR,
            pltpu.SemaphoreType.DMA((2,)),
            pltpu.SemaphoreType.DMA((2,)),
        ],
        compiler_params=pltpu.CompilerParams(collective_id=0),
    )(x, w_mat)
